# Initial kernel scaffold; baseline (speedup 1.0000x reference)
import jax
import jax.numpy as jnp
from jax.experimental import pallas as pl


def kernel(x, edge_index, edge_attr, W1, b1, W2, b2, W3, b3):
    raise NotImplementedError("write your pallas kernel here")


# speedup vs baseline: 4.4673x; 4.4673x over previous
import jax
import jax.numpy as jnp
from jax import lax
from jax.experimental import pallas as pl
from jax.experimental.pallas import tpu as pltpu
from jax.experimental.pallas import tpu_sc as plsc

F32 = jnp.float32
I32 = jnp.int32

NC, NS, L = 2, 16, 16
NW = NC * NS
LW = 128

N = 10000
NP = 10240
NPT = NP // NS
E = 160000
EP = 160768
EPW = EP // NW
EPT = EP // NS
NBUF = 4
D = 256
D2 = D // 2

NHW = 5056
ZPT = 2 * NHW // NS
NH3 = NP // 2
O3PT = NH3 // NS

_MESH = dict(core_axis_name="c", subcore_axis_name="s", num_cores=NC,
             num_subcores=NS)
_SC_PARAMS = pltpu.CompilerParams(needs_layout_passes=False)


def _ids():
    c = lax.axis_index("c")
    s = lax.axis_index("s")
    return c, s, s * NC + c


def _deg_body(dst_hbm, ew_hbm, parts_hbm, dstv, eww, buf, didx, degacc):
    c, s, w = _ids()
    iota = lax.iota(I32, L)
    zeros_i = jnp.zeros((L,), I32)
    for k in range(L):
        for j in range(LW // L):
            buf[k, pl.ds(j * L, L)] = jnp.zeros((L,), F32)
    for j in range(NPT // L):
        pltpu.sync_copy(buf, degacc.at[pl.ds(s * NPT + j * L, L)])
    base = w * EPW
    pltpu.sync_copy(dst_hbm.at[pl.ds(base, EPW)], dstv)
    pltpu.sync_copy(ew_hbm.at[pl.ds(base, EPW)], eww)
    plsc.subcore_barrier()

    def grp(g, carry):
        ew16 = eww[pl.ds(g * L, L)]
        dst16 = dstv[pl.ds(g * L, L)]
        plsc.store_scatter(buf, [iota, zeros_i], ew16)
        didx[...] = dst16
        pltpu.sync_copy(buf, degacc.at[plsc.Indices(didx)], add=True)
        return carry

    lax.fori_loop(0, EPW // L, grp, 0)
    plsc.subcore_barrier()
    nb = s * NPT
    pltpu.sync_copy(degacc.at[pl.ds(nb, NPT)],
                    parts_hbm.at[c, pl.ds(nb, NPT)])


_deg_call = pl.kernel(
    _deg_body,
    out_type=jax.ShapeDtypeStruct((NC, NP, LW), F32),
    mesh=plsc.VectorSubcoreMesh(**_MESH),
    compiler_params=_SC_PARAMS,
    scratch_types=[
        pltpu.VMEM((EPW,), I32),
        pltpu.VMEM((EPW,), F32),
        pltpu.VMEM((L, LW), F32),
        pltpu.VMEM((L,), I32),
        pltpu.VMEM_SHARED((NP, LW), F32),
    ],
)


def _norm_body(dis_hbm, src_hbm, dst_hbm, ew_hbm, norm_hbm,
               disfull, srcv, dstv, eww, normbuf):
    c, s, w = _ids()
    pltpu.sync_copy(dis_hbm, disfull)
    base = w * EPW
    pltpu.sync_copy(src_hbm.at[pl.ds(base, EPW)], srcv)
    pltpu.sync_copy(dst_hbm.at[pl.ds(base, EPW)], dstv)
    pltpu.sync_copy(ew_hbm.at[pl.ds(base, EPW)], eww)

    def grp(g, carry):
        sl = pl.ds(g * L, L)
        ds_ = plsc.load_gather(disfull, [srcv[sl]])
        dd_ = plsc.load_gather(disfull, [dstv[sl]])
        normbuf[sl] = ds_ * eww[sl] * dd_
        return carry

    lax.fori_loop(0, EPW // L, grp, 0)
    pltpu.sync_copy(normbuf, norm_hbm.at[pl.ds(base, EPW)])


_norm_call = pl.kernel(
    _norm_body,
    out_type=jax.ShapeDtypeStruct((EP,), F32),
    mesh=plsc.VectorSubcoreMesh(**_MESH),
    compiler_params=_SC_PARAMS,
    scratch_types=[
        pltpu.VMEM((NP,), F32),
        pltpu.VMEM((EPW,), I32),
        pltpu.VMEM((EPW,), I32),
        pltpu.VMEM((EPW,), F32),
        pltpu.VMEM((EPW,), F32),
    ],
)


def _agg_body(h_hbm, src_hbm, dst_hbm, norm_hbm, out_hbm,
              srcv, dstv, normv, rows0, rows1, rows2, rows3,
              gidx0, gidx1, gidx2, gidx3, sidx, zrows, acc, sem):
    c, s, w = _ids()
    rows = (rows0, rows1, rows2, rows3)
    gidx = (gidx0, gidx1, gidx2, gidx3)
    for k in range(L):
        for j in range(D2 // L):
            zrows[k, pl.ds(j * L, L)] = jnp.zeros((L,), F32)
    ab = s * ZPT
    for j in range(ZPT // L):
        pltpu.sync_copy(zrows, acc.at[pl.ds(ab + j * L, L)])
    pltpu.sync_copy(zrows.at[pl.ds(0, 8)],
                    acc.at[pl.ds(ab + (ZPT // L) * L, 8)])
    eb = s * EPT
    pltpu.sync_copy(src_hbm.at[pl.ds(eb, EPT)], srcv)
    pltpu.sync_copy(dst_hbm.at[pl.ds(eb, EPT)], dstv)
    pltpu.sync_copy(norm_hbm.at[pl.ds(eb, EPT)], normv)
    plsc.subcore_barrier()

    half_base = c * NHW

    def do_group(g, rbuf):
        for k in range(L):
            nk = plsc.load_gather(normv, [jnp.full((L,), g * L + k, I32)])
            for j in range(D2 // L):
                sl = pl.ds(j * L, L)
                rbuf[k, sl] = rbuf[k, sl] * nk
                rbuf[k + L, sl] = rbuf[k + L, sl] * nk
        dst16 = dstv[pl.ds(g * L, L)]
        dloc = dst16 - half_base
        oob = (dloc < 0) | (dloc >= NHW)
        sidx[pl.ds(0, L)] = jnp.where(oob, -1, 2 * dloc)
        sidx[pl.ds(L, L)] = jnp.where(oob, -1, 2 * dloc + 1)
        pltpu.sync_copy(rbuf, acc.at[plsc.Indices(sidx, ignored_value=-1)],
                        add=True)

    def body(gq, carry):
        g = gq * NBUF
        descs = []
        for b in range(NBUF):
            s16 = srcv[pl.ds((g + b) * L, L)]
            gidx[b][pl.ds(0, L)] = 2 * s16
            gidx[b][pl.ds(L, L)] = 2 * s16 + 1
            descs.append(
                pltpu.async_copy(h_hbm.at[plsc.Indices(gidx[b])], rows[b],
                                 sem))
        for d_ in descs:
            d_.wait()
        for b in range(NBUF):
            do_group(g + b, rows[b])
        return carry

    lax.fori_loop(0, EPT // L // NBUF, body, 0)
    plsc.subcore_barrier()
    ob = s * ZPT
    pltpu.sync_copy(acc.at[pl.ds(ob, ZPT)],
                    out_hbm.at[pl.ds(2 * half_base + ob, ZPT)])


_agg_call = pl.kernel(
    _agg_body,
    out_type=jax.ShapeDtypeStruct((2 * NP, D2), F32),
    mesh=plsc.VectorSubcoreMesh(**_MESH),
    compiler_params=_SC_PARAMS,
    scratch_types=[
        pltpu.VMEM((EPT,), I32),
        pltpu.VMEM((EPT,), I32),
        pltpu.VMEM((EPT,), F32),
        pltpu.VMEM((2 * L, D2), F32),
        pltpu.VMEM((2 * L, D2), F32),
        pltpu.VMEM((2 * L, D2), F32),
        pltpu.VMEM((2 * L, D2), F32),
        pltpu.VMEM((2 * L,), I32),
        pltpu.VMEM((2 * L,), I32),
        pltpu.VMEM((2 * L,), I32),
        pltpu.VMEM((2 * L,), I32),
        pltpu.VMEM((2 * L,), I32),
        pltpu.VMEM((L, D2), F32),
        pltpu.VMEM_SHARED((2 * NHW, D2), F32),
        pltpu.SemaphoreType.DMA,
    ],
)


def _agg3_body(h_hbm, src_hbm, dst_hbm, norm_hbm, out_hbm,
               srcv, dstv, normv, h3full, buf, sidx, acc):
    c, s, w = _ids()
    iota = lax.iota(I32, L)
    zeros_i = jnp.zeros((L,), I32)
    for k in range(L):
        for j in range(LW // L):
            buf[k, pl.ds(j * L, L)] = jnp.zeros((L,), F32)
    ab = s * O3PT
    for j in range(O3PT // L):
        pltpu.sync_copy(buf, acc.at[pl.ds(ab + j * L, L)])
    eb = s * EPT
    pltpu.sync_copy(src_hbm.at[pl.ds(eb, EPT)], srcv)
    pltpu.sync_copy(dst_hbm.at[pl.ds(eb, EPT)], dstv)
    pltpu.sync_copy(norm_hbm.at[pl.ds(eb, EPT)], normv)
    pltpu.sync_copy(h_hbm, h3full)
    plsc.subcore_barrier()

    half_base = c * NH3

    def grp(g, carry):
        sl = pl.ds(g * L, L)
        vals = plsc.load_gather(h3full, [srcv[sl]]) * normv[sl]
        plsc.store_scatter(buf, [iota, zeros_i], vals)
        dloc = dstv[sl] - half_base
        oob = (dloc < 0) | (dloc >= NH3)
        sidx[...] = jnp.where(oob, -1, dloc)
        pltpu.sync_copy(buf, acc.at[plsc.Indices(sidx, ignored_value=-1)],
                        add=True)
        return carry

    lax.fori_loop(0, EPT // L, grp, 0)
    plsc.subcore_barrier()
    ob = s * O3PT
    pltpu.sync_copy(acc.at[pl.ds(ob, O3PT)],
                    out_hbm.at[pl.ds(half_base + ob, O3PT)])


_agg3_call = pl.kernel(
    _agg3_body,
    out_type=jax.ShapeDtypeStruct((NP, LW), F32),
    mesh=plsc.VectorSubcoreMesh(**_MESH),
    compiler_params=_SC_PARAMS,
    scratch_types=[
        pltpu.VMEM((EPT,), I32),
        pltpu.VMEM((EPT,), I32),
        pltpu.VMEM((EPT,), F32),
        pltpu.VMEM((NP,), F32),
        pltpu.VMEM((L, LW), F32),
        pltpu.VMEM((L,), I32),
        pltpu.VMEM_SHARED((NH3, LW), F32),
    ],
)


_BM = 1024


def _mm_body(x_ref, w_ref, o_ref):
    o_ref[...] = jnp.dot(x_ref[...], w_ref[...], preferred_element_type=F32)


def _fused_body(a_ref, h_ref, d2_ref, b_ref, w_ref, o_ref):
    z = a_ref[...] + d2_ref[...] * h_ref[...] + b_ref[...]
    z = jnp.maximum(z, 0.0)
    o_ref[...] = jnp.dot(z, w_ref[...], preferred_element_type=F32)


def _disk_body(p0_ref, p1_ref, dis_ref, dis2_ref):
    d = 1.0 + p0_ref[...][:, 0:1] + p1_ref[...][:, 0:1]
    r = lax.rsqrt(d)
    dis_ref[...] = r
    dis2_ref[...] = r * r


def _fin_body(slab_ref, h3_ref, d2_ref, b3_ref, o_ref):
    o_ref[...] = (slab_ref[...][:, 0:1] + d2_ref[...] * h3_ref[...]
                  + b3_ref[...])


def _mm(xp, w):
    dout = w.shape[1]
    return pl.pallas_call(
        _mm_body,
        grid=(NP // _BM,),
        in_specs=[pl.BlockSpec((_BM, D), lambda i: (i, 0)),
                  pl.BlockSpec((D, dout), lambda i: (0, 0))],
        out_specs=pl.BlockSpec((_BM, dout), lambda i: (i, 0)),
        out_shape=jax.ShapeDtypeStruct((NP, dout), F32),
    )(xp, w)


def _fused(a, h, d2col, brow, w):
    dout = w.shape[1]
    return pl.pallas_call(
        _fused_body,
        grid=(NP // _BM,),
        in_specs=[pl.BlockSpec((_BM, D), lambda i: (i, 0)),
                  pl.BlockSpec((_BM, D), lambda i: (i, 0)),
                  pl.BlockSpec((_BM, 1), lambda i: (i, 0)),
                  pl.BlockSpec((1, D), lambda i: (0, 0)),
                  pl.BlockSpec((D, dout), lambda i: (0, 0))],
        out_specs=pl.BlockSpec((_BM, dout), lambda i: (i, 0)),
        out_shape=jax.ShapeDtypeStruct((NP, dout), F32),
    )(a, h, d2col, brow, w)


def _disk(p0, p1):
    return pl.pallas_call(
        _disk_body,
        grid=(NP // _BM,),
        in_specs=[pl.BlockSpec((_BM, LW), lambda i: (i, 0)),
                  pl.BlockSpec((_BM, LW), lambda i: (i, 0))],
        out_specs=(pl.BlockSpec((_BM, 1), lambda i: (i, 0)),
                   pl.BlockSpec((_BM, 1), lambda i: (i, 0))),
        out_shape=(jax.ShapeDtypeStruct((NP, 1), F32),
                   jax.ShapeDtypeStruct((NP, 1), F32)),
    )(p0, p1)


def _fin(slab, h3col, d2col, b3):
    return pl.pallas_call(
        _fin_body,
        grid=(NP // _BM,),
        in_specs=[pl.BlockSpec((_BM, LW), lambda i: (i, 0)),
                  pl.BlockSpec((_BM, 1), lambda i: (i, 0)),
                  pl.BlockSpec((_BM, 1), lambda i: (i, 0)),
                  pl.BlockSpec((1, 1), lambda i: (0, 0))],
        out_specs=pl.BlockSpec((_BM, 1), lambda i: (i, 0)),
        out_shape=jax.ShapeDtypeStruct((NP, 1), F32),
    )(slab, h3col, d2col, b3)


@jax.jit
def kernel(x, edge_index, edge_attr, W1, b1, W2, b2, W3, b3):
    src = edge_index[0]
    dst = edge_index[1]
    ew = edge_attr[:, 0]
    pe = EP - E
    srcp = jnp.concatenate([src, jnp.zeros((pe,), I32)])
    dstp = jnp.concatenate([dst, jnp.zeros((pe,), I32)])
    ewp = jnp.concatenate([ew, jnp.zeros((pe,), F32)])
    xp = jnp.concatenate([x, jnp.zeros((NP - N, D), F32)], axis=0)
    w3p = jnp.pad(W3, ((0, 0), (0, 127)))

    parts = _deg_call(dstp, ewp)
    discol, d2col = _disk(parts[0], parts[1])
    norm = _norm_call(discol[:, 0], srcp, dstp, ewp)

    h1 = _mm(xp, W1)
    a1 = _agg_call(h1.reshape(2 * NP, D2), srcp, dstp, norm).reshape(NP, D)
    h2 = _fused(a1, h1, d2col, b1[None, :], W2)
    a2 = _agg_call(h2.reshape(2 * NP, D2), srcp, dstp, norm).reshape(NP, D)
    h3w = _fused(a2, h2, d2col, b2[None, :], w3p)
    slab = _agg3_call(h3w[:, 0], srcp, dstp, norm)
    outp = _fin(slab, h3w[:, 0:1], d2col, b3[None, :])
    return outp[:N, 0]


# speedup vs baseline: 5.8186x; 1.3025x over previous
import jax
import jax.numpy as jnp
from jax import lax
from jax.experimental import pallas as pl
from jax.experimental.pallas import tpu as pltpu
from jax.experimental.pallas import tpu_sc as plsc

F32 = jnp.float32
I32 = jnp.int32

NC, NS, L = 2, 16, 16
NW = NC * NS
LW = 128

N = 10000
NP = 10240
NPT = NP // NS
E = 160000
EP = 160768
EPW = EP // NW
EPT = EP // NS
NBUF = 2
D = 256
D2 = D // 2

NHW = 5056
ZPT = 2 * NHW // NS
NH3 = NP // 2
O3PT = NH3 // NS

_MESH = dict(core_axis_name="c", subcore_axis_name="s", num_cores=NC,
             num_subcores=NS)
_SC_PARAMS = pltpu.CompilerParams(needs_layout_passes=False)


def _ids():
    c = lax.axis_index("c")
    s = lax.axis_index("s")
    return c, s, s * NC + c


def _deg_body(dst_hbm, ew_hbm, parts_hbm, dstv, eww, buf, didx, degacc):
    c, s, w = _ids()
    iota = lax.iota(I32, L)
    zeros_i = jnp.zeros((L,), I32)
    for k in range(L):
        for j in range(LW // L):
            buf[k, pl.ds(j * L, L)] = jnp.zeros((L,), F32)
    for j in range(NPT // L):
        pltpu.sync_copy(buf, degacc.at[pl.ds(s * NPT + j * L, L)])
    base = w * EPW
    pltpu.sync_copy(dst_hbm.at[pl.ds(base, EPW)], dstv)
    pltpu.sync_copy(ew_hbm.at[pl.ds(base, EPW)], eww)
    plsc.subcore_barrier()

    def grp(g, carry):
        ew16 = eww[pl.ds(g * L, L)]
        dst16 = dstv[pl.ds(g * L, L)]
        plsc.store_scatter(buf, [iota, zeros_i], ew16)
        didx[...] = dst16
        pltpu.sync_copy(buf, degacc.at[plsc.Indices(didx)], add=True)
        return carry

    lax.fori_loop(0, EPW // L, grp, 0)
    plsc.subcore_barrier()
    nb = s * NPT
    pltpu.sync_copy(degacc.at[pl.ds(nb, NPT)],
                    parts_hbm.at[c, pl.ds(nb, NPT)])


_deg_call = pl.kernel(
    _deg_body,
    out_type=jax.ShapeDtypeStruct((NC, NP, LW), F32),
    mesh=plsc.VectorSubcoreMesh(**_MESH),
    compiler_params=_SC_PARAMS,
    scratch_types=[
        pltpu.VMEM((EPW,), I32),
        pltpu.VMEM((EPW,), F32),
        pltpu.VMEM((L, LW), F32),
        pltpu.VMEM((L,), I32),
        pltpu.VMEM_SHARED((NP, LW), F32),
    ],
)


def _norm_body(dis_hbm, src_hbm, dst_hbm, ew_hbm, norm_hbm,
               disfull, srcv, dstv, eww, normbuf):
    c, s, w = _ids()
    pltpu.sync_copy(dis_hbm, disfull)
    base = w * EPW
    pltpu.sync_copy(src_hbm.at[pl.ds(base, EPW)], srcv)
    pltpu.sync_copy(dst_hbm.at[pl.ds(base, EPW)], dstv)
    pltpu.sync_copy(ew_hbm.at[pl.ds(base, EPW)], eww)

    def grp(g, carry):
        sl = pl.ds(g * L, L)
        ds_ = plsc.load_gather(disfull, [srcv[sl]])
        dd_ = plsc.load_gather(disfull, [dstv[sl]])
        normbuf[sl] = ds_ * eww[sl] * dd_
        return carry

    lax.fori_loop(0, EPW // L, grp, 0)
    pltpu.sync_copy(normbuf, norm_hbm.at[pl.ds(base, EPW)])


_norm_call = pl.kernel(
    _norm_body,
    out_type=jax.ShapeDtypeStruct((EP,), F32),
    mesh=plsc.VectorSubcoreMesh(**_MESH),
    compiler_params=_SC_PARAMS,
    scratch_types=[
        pltpu.VMEM((NP,), F32),
        pltpu.VMEM((EPW,), I32),
        pltpu.VMEM((EPW,), I32),
        pltpu.VMEM((EPW,), F32),
        pltpu.VMEM((EPW,), F32),
    ],
)


def _agg_body(h_hbm, src_hbm, dst_hbm, norm_hbm, out_hbm,
              srcv, dstv, normv,
              r0, r1, r4, r5,
              g0, g1, g4, g5,
              x0, x1, x4, x5,
              zrows, acc, gsem0, gsem1, ssem0, ssem1):
    c, s, w = _ids()
    sets = (((r0, r1), (g0, g1), (x0, x1), gsem0, ssem0),
            ((r4, r5), (g4, g5), (x4, x5), gsem1, ssem1))
    for k in range(L):
        for j in range(D2 // L):
            zrows[k, pl.ds(j * L, L)] = jnp.zeros((L,), F32)
    ab = s * ZPT
    for j in range(ZPT // L):
        pltpu.sync_copy(zrows, acc.at[pl.ds(ab + j * L, L)])
    pltpu.sync_copy(zrows.at[pl.ds(0, 8)],
                    acc.at[pl.ds(ab + (ZPT // L) * L, 8)])
    eb = s * EPT
    pltpu.sync_copy(src_hbm.at[pl.ds(eb, EPT)], srcv)
    pltpu.sync_copy(dst_hbm.at[pl.ds(eb, EPT)], dstv)
    pltpu.sync_copy(norm_hbm.at[pl.ds(eb, EPT)], normv)
    plsc.subcore_barrier()

    half_base = c * NHW
    NR = EPT // (L * NBUF)

    def issue_round(r, st):
        rws, gis, _, gsem, _ = st
        for b in range(NBUF):
            s16 = srcv[pl.ds((r * NBUF + b) * L, L)]
            gis[b][pl.ds(0, L)] = 2 * s16
            gis[b][pl.ds(L, L)] = 2 * s16 + 1
            pltpu.async_copy(h_hbm.at[plsc.Indices(gis[b])], rws[b], gsem)

    def drain_gathers(st):
        rws, gis, _, gsem, _ = st
        for b in range(NBUF):
            pltpu.make_async_copy(h_hbm.at[plsc.Indices(gis[b])], rws[b],
                                  gsem).wait()

    def process_round(r, st):
        rws, _, six, _, ssem = st
        for b in range(NBUF):
            g = r * NBUF + b
            rbuf = rws[b]
            for k in range(L):
                nk = plsc.load_gather(normv,
                                      [jnp.full((L,), g * L + k, I32)])
                for j in range(D2 // L):
                    sl = pl.ds(j * L, L)
                    rbuf[k, sl] = rbuf[k, sl] * nk
                    rbuf[k + L, sl] = rbuf[k + L, sl] * nk
            dst16 = dstv[pl.ds(g * L, L)]
            dloc = dst16 - half_base
            oob = (dloc < 0) | (dloc >= NHW)
            six[b][pl.ds(0, L)] = jnp.where(oob, -1, 2 * dloc)
            six[b][pl.ds(L, L)] = jnp.where(oob, -1, 2 * dloc + 1)
            pltpu.async_copy(
                rws[b], acc.at[plsc.Indices(six[b], ignored_value=-1)],
                ssem, add=True)

    def drain_scatters(st):
        rws, _, six, _, ssem = st
        for b in range(NBUF):
            pltpu.make_async_copy(
                rws[b], acc.at[plsc.Indices(six[b], ignored_value=-1)],
                ssem).wait()

    issue_round(0, sets[0])
    issue_round(1, sets[1])

    def body(r2, carry):
        for half, st in enumerate(sets):
            r = 2 * r2 + half
            drain_gathers(st)
            process_round(r, st)
            drain_scatters(st)

            @pl.when(r + 2 < NR)
            def _():
                issue_round(r + 2, st)

        return carry

    lax.fori_loop(0, NR // 2, body, 0)
    if NR % 2:
        drain_gathers(sets[0])
        process_round(NR - 1, sets[0])
        drain_scatters(sets[0])
    plsc.subcore_barrier()
    ob = s * ZPT
    pltpu.sync_copy(acc.at[pl.ds(ob, ZPT)],
                    out_hbm.at[pl.ds(2 * half_base + ob, ZPT)])


_agg_call = pl.kernel(
    _agg_body,
    out_type=jax.ShapeDtypeStruct((2 * NP, D2), F32),
    mesh=plsc.VectorSubcoreMesh(**_MESH),
    compiler_params=_SC_PARAMS,
    scratch_types=(
        [pltpu.VMEM((EPT,), I32),
         pltpu.VMEM((EPT,), I32),
         pltpu.VMEM((EPT,), F32)]
        + [pltpu.VMEM((2 * L, D2), F32) for _ in range(4)]
        + [pltpu.VMEM((2 * L,), I32) for _ in range(4)]
        + [pltpu.VMEM((2 * L,), I32) for _ in range(4)]
        + [pltpu.VMEM((L, D2), F32),
           pltpu.VMEM_SHARED((2 * NHW, D2), F32),
           pltpu.SemaphoreType.DMA, pltpu.SemaphoreType.DMA,
           pltpu.SemaphoreType.DMA, pltpu.SemaphoreType.DMA]
    ),
)


def _agg3_body(h_hbm, src_hbm, dst_hbm, norm_hbm, out_hbm,
               srcv, dstv, normv, h3full, buf, sidx, acc):
    c, s, w = _ids()
    iota = lax.iota(I32, L)
    zeros_i = jnp.zeros((L,), I32)
    for k in range(L):
        for j in range(LW // L):
            buf[k, pl.ds(j * L, L)] = jnp.zeros((L,), F32)
    ab = s * O3PT
    for j in range(O3PT // L):
        pltpu.sync_copy(buf, acc.at[pl.ds(ab + j * L, L)])
    eb = s * EPT
    pltpu.sync_copy(src_hbm.at[pl.ds(eb, EPT)], srcv)
    pltpu.sync_copy(dst_hbm.at[pl.ds(eb, EPT)], dstv)
    pltpu.sync_copy(norm_hbm.at[pl.ds(eb, EPT)], normv)
    pltpu.sync_copy(h_hbm, h3full)
    plsc.subcore_barrier()

    half_base = c * NH3

    def grp(g, carry):
        sl = pl.ds(g * L, L)
        vals = plsc.load_gather(h3full, [srcv[sl]]) * normv[sl]
        plsc.store_scatter(buf, [iota, zeros_i], vals)
        dloc = dstv[sl] - half_base
        oob = (dloc < 0) | (dloc >= NH3)
        sidx[...] = jnp.where(oob, -1, dloc)
        pltpu.sync_copy(buf, acc.at[plsc.Indices(sidx, ignored_value=-1)],
                        add=True)
        return carry

    lax.fori_loop(0, EPT // L, grp, 0)
    plsc.subcore_barrier()
    ob = s * O3PT
    pltpu.sync_copy(acc.at[pl.ds(ob, O3PT)],
                    out_hbm.at[pl.ds(half_base + ob, O3PT)])


_agg3_call = pl.kernel(
    _agg3_body,
    out_type=jax.ShapeDtypeStruct((NP, LW), F32),
    mesh=plsc.VectorSubcoreMesh(**_MESH),
    compiler_params=_SC_PARAMS,
    scratch_types=[
        pltpu.VMEM((EPT,), I32),
        pltpu.VMEM((EPT,), I32),
        pltpu.VMEM((EPT,), F32),
        pltpu.VMEM((NP,), F32),
        pltpu.VMEM((L, LW), F32),
        pltpu.VMEM((L,), I32),
        pltpu.VMEM_SHARED((NH3, LW), F32),
    ],
)


_BM = 1024


def _mm_body(x_ref, w_ref, o_ref):
    o_ref[...] = jnp.dot(x_ref[...], w_ref[...], preferred_element_type=F32)


def _fused_body(a_ref, h_ref, d2_ref, b_ref, w_ref, o_ref):
    z = a_ref[...] + d2_ref[...] * h_ref[...] + b_ref[...]
    z = jnp.maximum(z, 0.0)
    o_ref[...] = jnp.dot(z, w_ref[...], preferred_element_type=F32)


def _disk_body(p0_ref, p1_ref, dis_ref, dis2_ref):
    d = 1.0 + p0_ref[...][:, 0:1] + p1_ref[...][:, 0:1]
    r = lax.rsqrt(d)
    dis_ref[...] = r
    dis2_ref[...] = r * r


def _fin_body(slab_ref, h3_ref, d2_ref, b3_ref, o_ref):
    o_ref[...] = (slab_ref[...][:, 0:1] + d2_ref[...] * h3_ref[...]
                  + b3_ref[...])


def _mm(xp, w):
    dout = w.shape[1]
    return pl.pallas_call(
        _mm_body,
        grid=(NP // _BM,),
        in_specs=[pl.BlockSpec((_BM, D), lambda i: (i, 0)),
                  pl.BlockSpec((D, dout), lambda i: (0, 0))],
        out_specs=pl.BlockSpec((_BM, dout), lambda i: (i, 0)),
        out_shape=jax.ShapeDtypeStruct((NP, dout), F32),
    )(xp, w)


def _fused(a, h, d2col, brow, w):
    dout = w.shape[1]
    return pl.pallas_call(
        _fused_body,
        grid=(NP // _BM,),
        in_specs=[pl.BlockSpec((_BM, D), lambda i: (i, 0)),
                  pl.BlockSpec((_BM, D), lambda i: (i, 0)),
                  pl.BlockSpec((_BM, 1), lambda i: (i, 0)),
                  pl.BlockSpec((1, D), lambda i: (0, 0)),
                  pl.BlockSpec((D, dout), lambda i: (0, 0))],
        out_specs=pl.BlockSpec((_BM, dout), lambda i: (i, 0)),
        out_shape=jax.ShapeDtypeStruct((NP, dout), F32),
    )(a, h, d2col, brow, w)


def _disk(p0, p1):
    return pl.pallas_call(
        _disk_body,
        grid=(NP // _BM,),
        in_specs=[pl.BlockSpec((_BM, LW), lambda i: (i, 0)),
                  pl.BlockSpec((_BM, LW), lambda i: (i, 0))],
        out_specs=(pl.BlockSpec((_BM, 1), lambda i: (i, 0)),
                   pl.BlockSpec((_BM, 1), lambda i: (i, 0))),
        out_shape=(jax.ShapeDtypeStruct((NP, 1), F32),
                   jax.ShapeDtypeStruct((NP, 1), F32)),
    )(p0, p1)


def _fin(slab, h3col, d2col, b3):
    return pl.pallas_call(
        _fin_body,
        grid=(NP // _BM,),
        in_specs=[pl.BlockSpec((_BM, LW), lambda i: (i, 0)),
                  pl.BlockSpec((_BM, 1), lambda i: (i, 0)),
                  pl.BlockSpec((_BM, 1), lambda i: (i, 0)),
                  pl.BlockSpec((1, 1), lambda i: (0, 0))],
        out_specs=pl.BlockSpec((_BM, 1), lambda i: (i, 0)),
        out_shape=jax.ShapeDtypeStruct((NP, 1), F32),
    )(slab, h3col, d2col, b3)


@jax.jit
def kernel(x, edge_index, edge_attr, W1, b1, W2, b2, W3, b3):
    src = edge_index[0]
    dst = edge_index[1]
    ew = edge_attr[:, 0]
    pe = EP - E
    srcp = jnp.concatenate([src, jnp.zeros((pe,), I32)])
    dstp = jnp.concatenate([dst, jnp.zeros((pe,), I32)])
    ewp = jnp.concatenate([ew, jnp.zeros((pe,), F32)])
    xp = jnp.concatenate([x, jnp.zeros((NP - N, D), F32)], axis=0)
    w3p = jnp.pad(W3, ((0, 0), (0, 127)))

    parts = _deg_call(dstp, ewp)
    discol, d2col = _disk(parts[0], parts[1])
    norm = _norm_call(discol[:, 0], srcp, dstp, ewp)

    h1 = _mm(xp, W1)
    a1 = _agg_call(h1.reshape(2 * NP, D2), srcp, dstp, norm).reshape(NP, D)
    h2 = _fused(a1, h1, d2col, b1[None, :], W2)
    a2 = _agg_call(h2.reshape(2 * NP, D2), srcp, dstp, norm).reshape(NP, D)
    h3w = _fused(a2, h2, d2col, b2[None, :], w3p)
    slab = _agg3_call(h3w[:, 0], srcp, dstp, norm)
    outp = _fin(slab, h3w[:, 0:1], d2col, b3[None, :])
    return outp[:N, 0]


# speedup vs baseline: 6.1433x; 1.0558x over previous
import jax
import jax.numpy as jnp
from jax import lax
from jax.experimental import pallas as pl
from jax.experimental.pallas import tpu as pltpu
from jax.experimental.pallas import tpu_sc as plsc

F32 = jnp.float32
I32 = jnp.int32

NC, NS, L = 2, 16, 16
NW = NC * NS
LW = 128

N = 10000
NP = 10240
NPT = NP // NS
E = 160000
EP = 160768
EPW = EP // NW
EPT = EP // NS
NBUF = 2
D = 256
D2 = D // 2

NHW = 5056
ZPT = 2 * NHW // NS
NH3 = NP // 2
O3PT = NH3 // NS

_MESH = dict(core_axis_name="c", subcore_axis_name="s", num_cores=NC,
             num_subcores=NS)
_SC_PARAMS = pltpu.CompilerParams(needs_layout_passes=False)


def _ids():
    c = lax.axis_index("c")
    s = lax.axis_index("s")
    return c, s, s * NC + c


def _deg_body(dst_hbm, ew_hbm, parts_hbm, dstv, eww, bufa, bufb, didxa,
              didxb, degacc, sema, semb):
    c, s, w = _ids()
    iota = lax.iota(I32, L)
    zeros_i = jnp.zeros((L,), I32)
    bufs = (bufa, bufb)
    idxs = (didxa, didxb)
    sems = (sema, semb)
    for bb in bufs:
        for k in range(L):
            for j in range(LW // L):
                bb[k, pl.ds(j * L, L)] = jnp.zeros((L,), F32)
    for j in range(NPT // L):
        pltpu.sync_copy(bufa, degacc.at[pl.ds(s * NPT + j * L, L)])
    base = w * EPW
    pltpu.sync_copy(dst_hbm.at[pl.ds(base, EPW)], dstv)
    pltpu.sync_copy(ew_hbm.at[pl.ds(base, EPW)], eww)
    plsc.subcore_barrier()

    def build_issue(g, x):
        sl = pl.ds(g * L, L)
        plsc.store_scatter(bufs[x], [iota, zeros_i], eww[sl])
        idxs[x][...] = dstv[sl]
        pltpu.async_copy(bufs[x], degacc.at[plsc.Indices(idxs[x])], sems[x],
                         add=True)

    def wait_scatter(x):
        pltpu.make_async_copy(bufs[x], degacc.at[plsc.Indices(idxs[x])],
                              sems[x]).wait()

    build_issue(0, 0)
    build_issue(1, 1)

    def grp(g2, carry):
        for x in range(2):
            wait_scatter(x)
            build_issue(2 * g2 + x, x)
        return carry

    lax.fori_loop(1, EPW // L // 2, grp, 0)
    wait_scatter(0)
    wait_scatter(1)
    plsc.subcore_barrier()
    nb = s * NPT
    pltpu.sync_copy(degacc.at[pl.ds(nb, NPT)],
                    parts_hbm.at[c, pl.ds(nb, NPT)])


_deg_call = pl.kernel(
    _deg_body,
    out_type=jax.ShapeDtypeStruct((NC, NP, LW), F32),
    mesh=plsc.VectorSubcoreMesh(**_MESH),
    compiler_params=_SC_PARAMS,
    scratch_types=[
        pltpu.VMEM((EPW,), I32),
        pltpu.VMEM((EPW,), F32),
        pltpu.VMEM((L, LW), F32),
        pltpu.VMEM((L, LW), F32),
        pltpu.VMEM((L,), I32),
        pltpu.VMEM((L,), I32),
        pltpu.VMEM_SHARED((NP, LW), F32),
        pltpu.SemaphoreType.DMA,
        pltpu.SemaphoreType.DMA,
    ],
)


def _norm_body(dis_hbm, src_hbm, dst_hbm, ew_hbm, norm_hbm,
               disfull, srcv, dstv, eww, normbuf):
    c, s, w = _ids()
    pltpu.sync_copy(dis_hbm, disfull)
    base = w * EPW
    pltpu.sync_copy(src_hbm.at[pl.ds(base, EPW)], srcv)
    pltpu.sync_copy(dst_hbm.at[pl.ds(base, EPW)], dstv)
    pltpu.sync_copy(ew_hbm.at[pl.ds(base, EPW)], eww)

    def grp(g, carry):
        sl = pl.ds(g * L, L)
        ds_ = plsc.load_gather(disfull, [srcv[sl]])
        dd_ = plsc.load_gather(disfull, [dstv[sl]])
        normbuf[sl] = ds_ * eww[sl] * dd_
        return carry

    lax.fori_loop(0, EPW // L, grp, 0)
    pltpu.sync_copy(normbuf, norm_hbm.at[pl.ds(base, EPW)])


_norm_call = pl.kernel(
    _norm_body,
    out_type=jax.ShapeDtypeStruct((EP,), F32),
    mesh=plsc.VectorSubcoreMesh(**_MESH),
    compiler_params=_SC_PARAMS,
    scratch_types=[
        pltpu.VMEM((NP,), F32),
        pltpu.VMEM((EPW,), I32),
        pltpu.VMEM((EPW,), I32),
        pltpu.VMEM((EPW,), F32),
        pltpu.VMEM((EPW,), F32),
    ],
)


def _agg_body(h_hbm, src_hbm, dst_hbm, norm_hbm, out_hbm,
              srcv, dstv, normv,
              r0, r1, r4, r5,
              g0, g1, g4, g5,
              x0, x1, x4, x5,
              zrows, acc, gsem0, gsem1, ssem0, ssem1):
    c, s, w = _ids()
    sets = (((r0, r1), (g0, g1), (x0, x1), gsem0, ssem0),
            ((r4, r5), (g4, g5), (x4, x5), gsem1, ssem1))
    for k in range(L):
        for j in range(D2 // L):
            zrows[k, pl.ds(j * L, L)] = jnp.zeros((L,), F32)
    ab = s * ZPT
    for j in range(ZPT // L):
        pltpu.sync_copy(zrows, acc.at[pl.ds(ab + j * L, L)])
    pltpu.sync_copy(zrows.at[pl.ds(0, 8)],
                    acc.at[pl.ds(ab + (ZPT // L) * L, 8)])
    eb = s * EPT
    pltpu.sync_copy(src_hbm.at[pl.ds(eb, EPT)], srcv)
    pltpu.sync_copy(dst_hbm.at[pl.ds(eb, EPT)], dstv)
    pltpu.sync_copy(norm_hbm.at[pl.ds(eb, EPT)], normv)
    plsc.subcore_barrier()

    half_base = c * NHW
    NR = EPT // (L * NBUF)

    def issue_round(r, st):
        rws, gis, _, gsem, _ = st
        for b in range(NBUF):
            s16 = srcv[pl.ds((r * NBUF + b) * L, L)]
            gis[b][pl.ds(0, L)] = 2 * s16
            gis[b][pl.ds(L, L)] = 2 * s16 + 1
            pltpu.async_copy(h_hbm.at[plsc.Indices(gis[b])], rws[b], gsem)

    def drain_gathers(st):
        rws, gis, _, gsem, _ = st
        for b in range(NBUF):
            pltpu.make_async_copy(h_hbm.at[plsc.Indices(gis[b])], rws[b],
                                  gsem).wait()

    def process_round(r, st):
        rws, _, six, _, ssem = st
        for b in range(NBUF):
            g = r * NBUF + b
            rbuf = rws[b]
            for k in range(L):
                nk = plsc.load_gather(normv,
                                      [jnp.full((L,), g * L + k, I32)])
                for j in range(D2 // L):
                    sl = pl.ds(j * L, L)
                    rbuf[k, sl] = rbuf[k, sl] * nk
                    rbuf[k + L, sl] = rbuf[k + L, sl] * nk
            dst16 = dstv[pl.ds(g * L, L)]
            dloc = dst16 - half_base
            oob = (dloc < 0) | (dloc >= NHW)
            six[b][pl.ds(0, L)] = jnp.where(oob, -1, 2 * dloc)
            six[b][pl.ds(L, L)] = jnp.where(oob, -1, 2 * dloc + 1)
            pltpu.async_copy(
                rws[b], acc.at[plsc.Indices(six[b], ignored_value=-1)],
                ssem, add=True)

    def drain_scatters(st):
        rws, _, six, _, ssem = st
        for b in range(NBUF):
            pltpu.make_async_copy(
                rws[b], acc.at[plsc.Indices(six[b], ignored_value=-1)],
                ssem).wait()

    issue_round(0, sets[0])
    issue_round(1, sets[1])

    def body(r2, carry):
        for half, st in enumerate(sets):
            r = 2 * r2 + half
            drain_gathers(st)
            process_round(r, st)
            drain_scatters(st)

            @pl.when(r + 2 < NR)
            def _():
                issue_round(r + 2, st)

        return carry

    lax.fori_loop(0, NR // 2, body, 0)
    if NR % 2:
        drain_gathers(sets[0])
        process_round(NR - 1, sets[0])
        drain_scatters(sets[0])
    plsc.subcore_barrier()
    ob = s * ZPT
    pltpu.sync_copy(acc.at[pl.ds(ob, ZPT)],
                    out_hbm.at[pl.ds(2 * half_base + ob, ZPT)])


_agg_call = pl.kernel(
    _agg_body,
    out_type=jax.ShapeDtypeStruct((2 * NP, D2), F32),
    mesh=plsc.VectorSubcoreMesh(**_MESH),
    compiler_params=_SC_PARAMS,
    scratch_types=(
        [pltpu.VMEM((EPT,), I32),
         pltpu.VMEM((EPT,), I32),
         pltpu.VMEM((EPT,), F32)]
        + [pltpu.VMEM((2 * L, D2), F32) for _ in range(4)]
        + [pltpu.VMEM((2 * L,), I32) for _ in range(4)]
        + [pltpu.VMEM((2 * L,), I32) for _ in range(4)]
        + [pltpu.VMEM((L, D2), F32),
           pltpu.VMEM_SHARED((2 * NHW, D2), F32),
           pltpu.SemaphoreType.DMA, pltpu.SemaphoreType.DMA,
           pltpu.SemaphoreType.DMA, pltpu.SemaphoreType.DMA]
    ),
)


def _agg3_body(h_hbm, src_hbm, dst_hbm, norm_hbm, out_hbm,
               srcv, dstv, normv, h3full, bufa, bufb, sidxa, sidxb, acc,
               sema, semb):
    c, s, w = _ids()
    iota = lax.iota(I32, L)
    zeros_i = jnp.zeros((L,), I32)
    bufs = (bufa, bufb)
    idxs = (sidxa, sidxb)
    sems = (sema, semb)
    for bb in bufs:
        for k in range(L):
            for j in range(LW // L):
                bb[k, pl.ds(j * L, L)] = jnp.zeros((L,), F32)
    ab = s * O3PT
    for j in range(O3PT // L):
        pltpu.sync_copy(bufa, acc.at[pl.ds(ab + j * L, L)])
    eb = s * EPT
    pltpu.sync_copy(src_hbm.at[pl.ds(eb, EPT)], srcv)
    pltpu.sync_copy(dst_hbm.at[pl.ds(eb, EPT)], dstv)
    pltpu.sync_copy(norm_hbm.at[pl.ds(eb, EPT)], normv)
    pltpu.sync_copy(h_hbm, h3full)
    plsc.subcore_barrier()

    half_base = c * NH3

    def build_issue(g, x):
        sl = pl.ds(g * L, L)
        vals = plsc.load_gather(h3full, [srcv[sl]]) * normv[sl]
        plsc.store_scatter(bufs[x], [iota, zeros_i], vals)
        dloc = dstv[sl] - half_base
        oob = (dloc < 0) | (dloc >= NH3)
        idxs[x][...] = jnp.where(oob, -1, dloc)
        pltpu.async_copy(bufs[x],
                         acc.at[plsc.Indices(idxs[x], ignored_value=-1)],
                         sems[x], add=True)

    def wait_scatter(x):
        pltpu.make_async_copy(
            bufs[x], acc.at[plsc.Indices(idxs[x], ignored_value=-1)],
            sems[x]).wait()

    build_issue(0, 0)
    build_issue(1, 1)

    def grp(g2, carry):
        for x in range(2):
            wait_scatter(x)
            build_issue(2 * g2 + x, x)
        return carry

    lax.fori_loop(1, EPT // L // 2, grp, 0)
    wait_scatter(0)
    wait_scatter(1)
    plsc.subcore_barrier()
    ob = s * O3PT
    pltpu.sync_copy(acc.at[pl.ds(ob, O3PT)],
                    out_hbm.at[pl.ds(half_base + ob, O3PT)])


_agg3_call = pl.kernel(
    _agg3_body,
    out_type=jax.ShapeDtypeStruct((NP, LW), F32),
    mesh=plsc.VectorSubcoreMesh(**_MESH),
    compiler_params=_SC_PARAMS,
    scratch_types=[
        pltpu.VMEM((EPT,), I32),
        pltpu.VMEM((EPT,), I32),
        pltpu.VMEM((EPT,), F32),
        pltpu.VMEM((NP,), F32),
        pltpu.VMEM((L, LW), F32),
        pltpu.VMEM((L, LW), F32),
        pltpu.VMEM((L,), I32),
        pltpu.VMEM((L,), I32),
        pltpu.VMEM_SHARED((NH3, LW), F32),
        pltpu.SemaphoreType.DMA,
        pltpu.SemaphoreType.DMA,
    ],
)


_BM = 1024


def _mm_body(x_ref, w_ref, o_ref):
    o_ref[...] = jnp.dot(x_ref[...], w_ref[...], preferred_element_type=F32)


def _fused_body(a_ref, h_ref, d2_ref, b_ref, w_ref, o_ref):
    z = a_ref[...] + d2_ref[...] * h_ref[...] + b_ref[...]
    z = jnp.maximum(z, 0.0)
    o_ref[...] = jnp.dot(z, w_ref[...], preferred_element_type=F32)


def _disk_body(p0_ref, p1_ref, dis_ref, dis2_ref):
    d = 1.0 + p0_ref[...][:, 0:1] + p1_ref[...][:, 0:1]
    r = lax.rsqrt(d)
    dis_ref[...] = r
    dis2_ref[...] = r * r


def _fin_body(slab_ref, h3_ref, d2_ref, b3_ref, o_ref):
    o_ref[...] = (slab_ref[...][:, 0:1] + d2_ref[...] * h3_ref[...]
                  + b3_ref[...])


def _mm(xp, w):
    dout = w.shape[1]
    return pl.pallas_call(
        _mm_body,
        grid=(NP // _BM,),
        in_specs=[pl.BlockSpec((_BM, D), lambda i: (i, 0)),
                  pl.BlockSpec((D, dout), lambda i: (0, 0))],
        out_specs=pl.BlockSpec((_BM, dout), lambda i: (i, 0)),
        out_shape=jax.ShapeDtypeStruct((NP, dout), F32),
    )(xp, w)


def _fused(a, h, d2col, brow, w):
    dout = w.shape[1]
    return pl.pallas_call(
        _fused_body,
        grid=(NP // _BM,),
        in_specs=[pl.BlockSpec((_BM, D), lambda i: (i, 0)),
                  pl.BlockSpec((_BM, D), lambda i: (i, 0)),
                  pl.BlockSpec((_BM, 1), lambda i: (i, 0)),
                  pl.BlockSpec((1, D), lambda i: (0, 0)),
                  pl.BlockSpec((D, dout), lambda i: (0, 0))],
        out_specs=pl.BlockSpec((_BM, dout), lambda i: (i, 0)),
        out_shape=jax.ShapeDtypeStruct((NP, dout), F32),
    )(a, h, d2col, brow, w)


def _disk(p0, p1):
    return pl.pallas_call(
        _disk_body,
        grid=(NP // _BM,),
        in_specs=[pl.BlockSpec((_BM, LW), lambda i: (i, 0)),
                  pl.BlockSpec((_BM, LW), lambda i: (i, 0))],
        out_specs=(pl.BlockSpec((_BM, 1), lambda i: (i, 0)),
                   pl.BlockSpec((_BM, 1), lambda i: (i, 0))),
        out_shape=(jax.ShapeDtypeStruct((NP, 1), F32),
                   jax.ShapeDtypeStruct((NP, 1), F32)),
    )(p0, p1)


def _fin(slab, h3col, d2col, b3):
    return pl.pallas_call(
        _fin_body,
        grid=(NP // _BM,),
        in_specs=[pl.BlockSpec((_BM, LW), lambda i: (i, 0)),
                  pl.BlockSpec((_BM, 1), lambda i: (i, 0)),
                  pl.BlockSpec((_BM, 1), lambda i: (i, 0)),
                  pl.BlockSpec((1, 1), lambda i: (0, 0))],
        out_specs=pl.BlockSpec((_BM, 1), lambda i: (i, 0)),
        out_shape=jax.ShapeDtypeStruct((NP, 1), F32),
    )(slab, h3col, d2col, b3)


@jax.jit
def kernel(x, edge_index, edge_attr, W1, b1, W2, b2, W3, b3):
    src = edge_index[0]
    dst = edge_index[1]
    ew = edge_attr[:, 0]
    pe = EP - E
    srcp = jnp.concatenate([src, jnp.zeros((pe,), I32)])
    dstp = jnp.concatenate([dst, jnp.zeros((pe,), I32)])
    ewp = jnp.concatenate([ew, jnp.zeros((pe,), F32)])
    xp = jnp.concatenate([x, jnp.zeros((NP - N, D), F32)], axis=0)
    w3p = jnp.pad(W3, ((0, 0), (0, 127)))

    parts = _deg_call(dstp, ewp)
    discol, d2col = _disk(parts[0], parts[1])
    norm = _norm_call(discol[:, 0], srcp, dstp, ewp)

    h1 = _mm(xp, W1)
    a1 = _agg_call(h1.reshape(2 * NP, D2), srcp, dstp, norm).reshape(NP, D)
    h2 = _fused(a1, h1, d2col, b1[None, :], W2)
    a2 = _agg_call(h2.reshape(2 * NP, D2), srcp, dstp, norm).reshape(NP, D)
    h3w = _fused(a2, h2, d2col, b2[None, :], w3p)
    slab = _agg3_call(h3w[:, 0], srcp, dstp, norm)
    outp = _fin(slab, h3w[:, 0:1], d2col, b3[None, :])
    return outp[:N, 0]
